# baseline (speedup 1.0000x reference)
import functools

import jax
import jax.numpy as jnp
from jax import lax
from jax.experimental import pallas as pl
from jax.experimental.pallas import tpu as pltpu
from jax.experimental.pallas import tpu_sc as plsc

_LN2 = 0.6931471805599453

N = 10000
NBR = 32
D = 128
S = 16
F = 64
CUTOFF = 0.5
E = N * NBR
EH = E // 2
ZR = 1024
NT = N + ZR

NC, NS = 2, 16
NW = NC * NS
EPW = E // NW
CH = 400


def _ssp(v):
    return jnp.maximum(v, 0.0) + jnp.log1p(jnp.exp(-jnp.abs(v))) - _LN2


def _in2f_body(x_ref, w_ref, o_ref):
    o_ref[0:N, :] = jnp.dot(
        x_ref[...], w_ref[...],
        preferred_element_type=jnp.float32).astype(jnp.bfloat16)
    o_ref[N:NT, :] = jnp.zeros((NT - N, F), jnp.bfloat16)


def _in2f(x2, w_in):
    return pl.pallas_call(
        _in2f_body,
        out_shape=jax.ShapeDtypeStruct((NT, F), jnp.bfloat16),
    )(x2, w_in)


def _sc_gather(table, idx):
    mesh = plsc.VectorSubcoreMesh(core_axis_name="c", subcore_axis_name="s")

    @functools.partial(
        pl.kernel,
        out_type=jax.ShapeDtypeStruct((EH, D), jnp.bfloat16),
        mesh=mesh,
        scratch_types=[
            pltpu.VMEM((EPW,), jnp.int32),
            pltpu.VMEM((CH, F), jnp.bfloat16),
            pltpu.VMEM((CH, F), jnp.bfloat16),
            pltpu.SemaphoreType.DMA,
            pltpu.SemaphoreType.DMA,
        ],
        compiler_params=pltpu.CompilerParams(use_tc_tiling_on_sc=False),
    )
    def k(table_hbm, idx_hbm, out_hbm, idx_v, rows_a, rows_b, sem_g, sem_w):
        wid = lax.axis_index("s") * NC + lax.axis_index("c")
        half = wid // 16
        prow = (wid % 16) * EPW
        pltpu.sync_copy(idx_hbm.at[pl.ds(wid * EPW, EPW)], idx_v)
        bufs = (rows_a, rows_b)
        nch = EPW // CH

        def gather(i):
            return pltpu.async_copy(
                table_hbm.at[idx_v.at[pl.ds(i * CH, CH)]],
                bufs[i % 2], sem_g)

        def write(i):
            return pltpu.async_copy(
                bufs[i % 2],
                out_hbm.at[pl.ds(prow + i * CH, CH), pl.ds(half * F, F)],
                sem_w)

        g = gather(0)
        w_prev = None
        for i in range(nch):
            g.wait()
            if w_prev is not None:
                w_prev.wait()
            if i + 1 < nch:
                g = gather(i + 1)
            w = write(i)
            w_prev = w if i + 1 < nch else None
            if i + 1 >= nch:
                w.wait()

    return k(table, idx)


def _fused_body(fa_ref, fb_ref, g_ref,
                w1l_ref, w1r_ref, b1_ref, w2_ref, b2_ref,
                wol_ref, wor_ref, bo_ref, wd_ref, bd_ref, o_ref, *, t):
    fa = fa_ref[...].astype(jnp.float32)
    fb = fb_ref[...].astype(jnp.float32)
    h = (jnp.dot(fa, w1l_ref[...], preferred_element_type=jnp.float32)
         + jnp.dot(fb, w1r_ref[...], preferred_element_type=jnp.float32)
         + b1_ref[...])
    w = jnp.dot(_ssp(h), w2_ref[...],
                preferred_element_type=jnp.float32) + b2_ref[...]
    g = g_ref[...].astype(jnp.float32)
    p = (g * w).reshape(t, NBR, D).sum(axis=1)
    ya = jnp.dot(p, wol_ref[...], preferred_element_type=jnp.float32)
    yb = jnp.dot(p, wor_ref[...], preferred_element_type=jnp.float32)
    ya = _ssp(ya + bo_ref[...])
    yb = _ssp(yb + bo_ref[...])
    va = jnp.dot(ya, wd_ref[...],
                 preferred_element_type=jnp.float32) + bd_ref[...]
    vb = jnp.dot(yb, wd_ref[...],
                 preferred_element_type=jnp.float32) + bd_ref[...]
    o_ref[...] = jnp.stack([va, vb])


def _fused(f2, gp, w1l, w1r, b1p, w2p, b2p, wol, wor, bo, wd, bd, t):
    ep = t * NBR
    grid = (N // 2) // t
    full = lambda i: (0, 0)
    return pl.pallas_call(
        functools.partial(_fused_body, t=t),
        grid=(grid,),
        in_specs=[
            pl.BlockSpec((ep, S), lambda i: (i, 0)),
            pl.BlockSpec((ep, S), lambda i, g=grid: (i + g, 0)),
            pl.BlockSpec((ep, D), lambda i: (i, 0)),
            pl.BlockSpec((S, D), full),
            pl.BlockSpec((S, D), full),
            pl.BlockSpec((1, D), full),
            pl.BlockSpec((D, D), full),
            pl.BlockSpec((1, D), full),
            pl.BlockSpec((D, D), full),
            pl.BlockSpec((D, D), full),
            pl.BlockSpec((1, D), full),
            pl.BlockSpec((D, D), full),
            pl.BlockSpec((1, D), full),
        ],
        out_specs=pl.BlockSpec((2, t, D), lambda i: (0, i, 0)),
        out_shape=jax.ShapeDtypeStruct((2, N // 2, D), jnp.float32),
    )(f2, f2, gp, w1l, w1r, b1p, w2p, b2p, wol, wor, bo, wd, bd)


def kernel(x, r_ij, neighbors, neighbor_mask, f_ij,
           W1, b1, W2, b2, W_in, W_out, b_out, W_d, b_d):
    x2 = x.reshape(N, D)
    keep = (r_ij <= CUTOFF) & (neighbor_mask != 0)
    ii = lax.broadcasted_iota(jnp.int32, (1, N, NBR), 1)
    jj = lax.broadcasted_iota(jnp.int32, (1, N, NBR), 2)
    spread = N + ((ii * NBR + jj) & (ZR - 1))
    idx = jnp.where(keep, neighbors.astype(jnp.int32), spread).reshape(E)
    f2 = f_ij.astype(jnp.bfloat16).reshape(E, S)

    zs = jnp.zeros((S, F), jnp.float32)
    w1l = jnp.concatenate([W1, zs], axis=1)
    w1r = jnp.concatenate([zs, W1], axis=1)
    zf = jnp.zeros((F, F), jnp.float32)
    w2p = jnp.concatenate(
        [jnp.concatenate([W2, zf], axis=1), jnp.concatenate([zf, W2], axis=1)],
        axis=0)
    b1p = jnp.concatenate([b1, b1]).reshape(1, D)
    b2p = jnp.concatenate([b2, b2]).reshape(1, D)
    zd = jnp.zeros((F, D), jnp.float32)
    wol = jnp.concatenate([W_out, zd], axis=0)
    wor = jnp.concatenate([zd, W_out], axis=0)

    y = _in2f(x2, W_in)
    gp = _sc_gather(y, idx)
    v = _fused(f2, gp, w1l, w1r, b1p, w2p, b2p, wol, wor,
               b_out.reshape(1, D), W_d, b_d.reshape(1, D), t=200)
    return v.reshape(1, N, D)


# speedup vs baseline: 1.0236x; 1.0236x over previous
import functools

import jax
import jax.numpy as jnp
from jax import lax
from jax.experimental import pallas as pl
from jax.experimental.pallas import tpu as pltpu
from jax.experimental.pallas import tpu_sc as plsc

_LN2 = 0.6931471805599453

N = 10000
NBR = 32
D = 128
S = 16
F = 64
CUTOFF = 0.5
E = N * NBR
EH = E // 2
ZR = 1024
NT = N + ZR

NC, NS = 2, 16
NW = NC * NS
EPW = E // NW
CH = 1000


def _ssp(v):
    return jnp.maximum(v, 0.0) + jnp.log1p(jnp.exp(-jnp.abs(v))) - _LN2


def _in2f_body(x_ref, w_ref, o_ref):
    o_ref[0:N, :] = jnp.dot(
        x_ref[...], w_ref[...],
        preferred_element_type=jnp.float32).astype(jnp.bfloat16)
    o_ref[N:NT, :] = jnp.zeros((NT - N, F), jnp.bfloat16)


def _in2f(x2, w_in):
    return pl.pallas_call(
        _in2f_body,
        out_shape=jax.ShapeDtypeStruct((NT, F), jnp.bfloat16),
    )(x2, w_in)


def _sc_gather(table, idx):
    mesh = plsc.VectorSubcoreMesh(core_axis_name="c", subcore_axis_name="s")

    @functools.partial(
        pl.kernel,
        out_type=jax.ShapeDtypeStruct((EH, D), jnp.bfloat16),
        mesh=mesh,
        scratch_types=[
            pltpu.VMEM((EPW,), jnp.int32),
            pltpu.VMEM((CH, F), jnp.bfloat16),
            pltpu.VMEM((CH, F), jnp.bfloat16),
            pltpu.SemaphoreType.DMA,
            pltpu.SemaphoreType.DMA,
        ],
        compiler_params=pltpu.CompilerParams(use_tc_tiling_on_sc=False),
    )
    def k(table_hbm, idx_hbm, out_hbm, idx_v, rows_a, rows_b, sem_g, sem_w):
        wid = lax.axis_index("s") * NC + lax.axis_index("c")
        half = wid // 16
        prow = (wid % 16) * EPW
        pltpu.sync_copy(idx_hbm.at[pl.ds(wid * EPW, EPW)], idx_v)
        bufs = (rows_a, rows_b)
        nch = EPW // CH

        def gather(i):
            return pltpu.async_copy(
                table_hbm.at[idx_v.at[pl.ds(i * CH, CH)]],
                bufs[i % 2], sem_g)

        def write(i):
            return pltpu.async_copy(
                bufs[i % 2],
                out_hbm.at[pl.ds(prow + i * CH, CH), pl.ds(half * F, F)],
                sem_w)

        g = gather(0)
        w_prev = None
        for i in range(nch):
            g.wait()
            if w_prev is not None:
                w_prev.wait()
            if i + 1 < nch:
                g = gather(i + 1)
            w = write(i)
            w_prev = w if i + 1 < nch else None
            if i + 1 >= nch:
                w.wait()

    return k(table, idx)


def _fused_body(fa_ref, fb_ref, g_ref,
                w1l_ref, w1r_ref, b1_ref, w2_ref, b2_ref,
                wol_ref, wor_ref, bo_ref, wd_ref, bd_ref, o_ref, *, t):
    h = (jnp.dot(fa_ref[...], w1l_ref[...], preferred_element_type=jnp.float32)
         + jnp.dot(fb_ref[...], w1r_ref[...],
                   preferred_element_type=jnp.float32)
         + b1_ref[...])
    w = jnp.dot(_ssp(h), w2_ref[...],
                preferred_element_type=jnp.float32) + b2_ref[...]
    g = g_ref[...].astype(jnp.float32)
    p = (g * w).reshape(t, NBR, D).sum(axis=1)
    ya = jnp.dot(p, wol_ref[...], preferred_element_type=jnp.float32)
    yb = jnp.dot(p, wor_ref[...], preferred_element_type=jnp.float32)
    ya = _ssp(ya + bo_ref[...])
    yb = _ssp(yb + bo_ref[...])
    va = jnp.dot(ya, wd_ref[...],
                 preferred_element_type=jnp.float32) + bd_ref[...]
    vb = jnp.dot(yb, wd_ref[...],
                 preferred_element_type=jnp.float32) + bd_ref[...]
    o_ref[...] = jnp.stack([va, vb])


def _fused(f2, gp, w1l, w1r, b1p, w2p, b2p, wol, wor, bo, wd, bd, t):
    ep = t * NBR
    grid = (N // 2) // t
    full = lambda i: (0, 0)
    return pl.pallas_call(
        functools.partial(_fused_body, t=t),
        grid=(grid,),
        in_specs=[
            pl.BlockSpec((ep, S), lambda i: (i, 0)),
            pl.BlockSpec((ep, S), lambda i, g=grid: (i + g, 0)),
            pl.BlockSpec((ep, D), lambda i: (i, 0)),
            pl.BlockSpec((S, D), full),
            pl.BlockSpec((S, D), full),
            pl.BlockSpec((1, D), full),
            pl.BlockSpec((D, D), full),
            pl.BlockSpec((1, D), full),
            pl.BlockSpec((D, D), full),
            pl.BlockSpec((D, D), full),
            pl.BlockSpec((1, D), full),
            pl.BlockSpec((D, D), full),
            pl.BlockSpec((1, D), full),
        ],
        out_specs=pl.BlockSpec((2, t, D), lambda i: (0, i, 0)),
        out_shape=jax.ShapeDtypeStruct((2, N // 2, D), jnp.float32),
    )(f2, f2, gp, w1l, w1r, b1p, w2p, b2p, wol, wor, bo, wd, bd)


def kernel(x, r_ij, neighbors, neighbor_mask, f_ij,
           W1, b1, W2, b2, W_in, W_out, b_out, W_d, b_d):
    x2 = x.reshape(N, D)
    keep = (r_ij <= CUTOFF) & (neighbor_mask != 0)
    ii = lax.broadcasted_iota(jnp.int32, (1, N, NBR), 1)
    jj = lax.broadcasted_iota(jnp.int32, (1, N, NBR), 2)
    spread = N + ((ii * NBR + jj) & (ZR - 1))
    idx = jnp.where(keep, neighbors.astype(jnp.int32), spread).reshape(E)
    f2 = f_ij.reshape(E, S)

    zs = jnp.zeros((S, F), jnp.float32)
    w1l = jnp.concatenate([W1, zs], axis=1)
    w1r = jnp.concatenate([zs, W1], axis=1)
    zf = jnp.zeros((F, F), jnp.float32)
    w2p = jnp.concatenate(
        [jnp.concatenate([W2, zf], axis=1), jnp.concatenate([zf, W2], axis=1)],
        axis=0)
    b1p = jnp.concatenate([b1, b1]).reshape(1, D)
    b2p = jnp.concatenate([b2, b2]).reshape(1, D)
    zd = jnp.zeros((F, D), jnp.float32)
    wol = jnp.concatenate([W_out, zd], axis=0)
    wor = jnp.concatenate([zd, W_out], axis=0)

    y = _in2f(x2, W_in)
    gp = _sc_gather(y, idx)
    v = _fused(f2, gp, w1l, w1r, b1p, w2p, b2p, wol, wor,
               b_out.reshape(1, D), W_d, b_d.reshape(1, D), t=200)
    return v.reshape(1, N, D)


# speedup vs baseline: 1.3693x; 1.3377x over previous
import functools

import jax
import jax.numpy as jnp
from jax import lax
from jax.experimental import pallas as pl
from jax.experimental.pallas import tpu as pltpu
from jax.experimental.pallas import tpu_sc as plsc

_LN2 = 0.6931471805599453

N = 10000
NBR = 32
D = 128
S = 16
F = 64
CUTOFF = 0.5
E = N * NBR
EH = E // 2
ZR = 1024
NT = N + ZR

NC, NS = 2, 16
NW = NC * NS
EPW = E // NW
CH = 400


def _ssp(v):
    return jnp.maximum(v, 0.0) + jnp.log1p(jnp.exp(-jnp.abs(v))) - _LN2


def _in2f_body(x_ref, w_ref, o_ref):
    o_ref[0:N, :] = jnp.dot(x_ref[...], w_ref[...],
                            preferred_element_type=jnp.float32)
    o_ref[N:NT, :] = jnp.zeros((NT - N, F), jnp.float32)


def _in2f(x2, w_in):
    return pl.pallas_call(
        _in2f_body,
        out_shape=jax.ShapeDtypeStruct((NT, F), jnp.float32),
    )(x2, w_in)


def _sc_gather(table, idx):
    mesh = plsc.VectorSubcoreMesh(core_axis_name="c", subcore_axis_name="s")

    @functools.partial(
        pl.kernel,
        out_type=jax.ShapeDtypeStruct((EH, D), jnp.float32),
        mesh=mesh,
        scratch_types=[
            pltpu.VMEM((EPW,), jnp.int32),
            pltpu.VMEM((CH, F), jnp.float32),
            pltpu.VMEM((CH, F), jnp.float32),
            pltpu.SemaphoreType.DMA,
            pltpu.SemaphoreType.DMA,
        ],
        compiler_params=pltpu.CompilerParams(use_tc_tiling_on_sc=False),
    )
    def k(table_hbm, idx_hbm, out_hbm, idx_v, rows_a, rows_b, sem_g, sem_w):
        wid = lax.axis_index("s") * NC + lax.axis_index("c")
        half = wid // 16
        prow = (wid % 16) * EPW
        pltpu.sync_copy(idx_hbm.at[pl.ds(wid * EPW, EPW)], idx_v)
        bufs = (rows_a, rows_b)
        nch = EPW // CH

        def gather(i):
            return pltpu.async_copy(
                table_hbm.at[idx_v.at[pl.ds(i * CH, CH)]],
                bufs[i % 2], sem_g)

        def write(i):
            return pltpu.async_copy(
                bufs[i % 2],
                out_hbm.at[pl.ds(prow + i * CH, CH), pl.ds(half * F, F)],
                sem_w)

        g = gather(0)
        w_prev = None
        for i in range(nch):
            g.wait()
            if w_prev is not None:
                w_prev.wait()
            if i + 1 < nch:
                g = gather(i + 1)
            w = write(i)
            w_prev = w if i + 1 < nch else None
            if i + 1 >= nch:
                w.wait()

    return k(table, idx)


def _filt_body(fa_ref, fb_ref, w1l_ref, w1r_ref, b1_ref, w2_ref, b2_ref,
               o_ref):
    h = (jnp.dot(fa_ref[...], w1l_ref[...], preferred_element_type=jnp.float32)
         + jnp.dot(fb_ref[...], w1r_ref[...],
                   preferred_element_type=jnp.float32)
         + b1_ref[...])
    o_ref[...] = jnp.dot(_ssp(h), w2_ref[...],
                         preferred_element_type=jnp.float32) + b2_ref[...]


def _filt(f2, w1l, w1r, b1p, w2p, b2p, t):
    ep = t * NBR
    grid = (N // 2) // t
    full = lambda i: (0, 0)
    return pl.pallas_call(
        _filt_body,
        grid=(grid,),
        in_specs=[
            pl.BlockSpec((ep, S), lambda i: (i, 0)),
            pl.BlockSpec((ep, S), lambda i, g=grid: (i + g, 0)),
            pl.BlockSpec((S, D), full),
            pl.BlockSpec((S, D), full),
            pl.BlockSpec((1, D), full),
            pl.BlockSpec((D, D), full),
            pl.BlockSpec((1, D), full),
        ],
        out_specs=pl.BlockSpec((ep, D), lambda i: (i, 0)),
        out_shape=jax.ShapeDtypeStruct((EH, D), jnp.float32),
    )(f2, f2, w1l, w1r, b1p, w2p, b2p)


def _tail_body(g_ref, w_ref, wol_ref, wor_ref, bo_ref, wd_ref, bd_ref,
               o_ref, *, t):
    p = (g_ref[...] * w_ref[...]).reshape(t, NBR, D).sum(axis=1)
    ya = jnp.dot(p, wol_ref[...], preferred_element_type=jnp.float32)
    yb = jnp.dot(p, wor_ref[...], preferred_element_type=jnp.float32)
    ya = _ssp(ya + bo_ref[...])
    yb = _ssp(yb + bo_ref[...])
    va = jnp.dot(ya, wd_ref[...],
                 preferred_element_type=jnp.float32) + bd_ref[...]
    vb = jnp.dot(yb, wd_ref[...],
                 preferred_element_type=jnp.float32) + bd_ref[...]
    o_ref[...] = jnp.stack([va, vb])


def _tail(gp, wp, wol, wor, bo, wd, bd, t):
    ep = t * NBR
    grid = (N // 2) // t
    full = lambda i: (0, 0)
    return pl.pallas_call(
        functools.partial(_tail_body, t=t),
        grid=(grid,),
        in_specs=[
            pl.BlockSpec((ep, D), lambda i: (i, 0)),
            pl.BlockSpec((ep, D), lambda i: (i, 0)),
            pl.BlockSpec((D, D), full),
            pl.BlockSpec((D, D), full),
            pl.BlockSpec((1, D), full),
            pl.BlockSpec((D, D), full),
            pl.BlockSpec((1, D), full),
        ],
        out_specs=pl.BlockSpec((2, t, D), lambda i: (0, i, 0)),
        out_shape=jax.ShapeDtypeStruct((2, N // 2, D), jnp.float32),
    )(gp, wp, wol, wor, bo, wd, bd)


def kernel(x, r_ij, neighbors, neighbor_mask, f_ij,
           W1, b1, W2, b2, W_in, W_out, b_out, W_d, b_d):
    x2 = x.reshape(N, D)
    keep = (r_ij <= CUTOFF) & (neighbor_mask != 0)
    ii = lax.broadcasted_iota(jnp.int32, (1, N, NBR), 1)
    jj = lax.broadcasted_iota(jnp.int32, (1, N, NBR), 2)
    spread = N + ((ii * NBR + jj) & (ZR - 1))
    idx = jnp.where(keep, neighbors.astype(jnp.int32), spread).reshape(E)
    f2 = f_ij.reshape(E, S)

    zs = jnp.zeros((S, F), jnp.float32)
    w1l = jnp.concatenate([W1, zs], axis=1)
    w1r = jnp.concatenate([zs, W1], axis=1)
    zf = jnp.zeros((F, F), jnp.float32)
    w2p = jnp.concatenate(
        [jnp.concatenate([W2, zf], axis=1), jnp.concatenate([zf, W2], axis=1)],
        axis=0)
    b1p = jnp.concatenate([b1, b1]).reshape(1, D)
    b2p = jnp.concatenate([b2, b2]).reshape(1, D)
    zd = jnp.zeros((F, D), jnp.float32)
    wol = jnp.concatenate([W_out, zd], axis=0)
    wor = jnp.concatenate([zd, W_out], axis=0)

    y = _in2f(x2, W_in)
    gp = _sc_gather(y, idx)
    wp = _filt(f2, w1l, w1r, b1p, w2p, b2p, t=200)
    v = _tail(gp, wp, wol, wor,
              b_out.reshape(1, D), W_d, b_d.reshape(1, D), t=200)
    return v.reshape(1, N, D)


# speedup vs baseline: 1.9311x; 1.4103x over previous
import functools

import jax
import jax.numpy as jnp
from jax import lax
from jax.experimental import pallas as pl
from jax.experimental.pallas import tpu as pltpu
from jax.experimental.pallas import tpu_sc as plsc

_LN2 = 0.6931471805599453

N = 10000
NBR = 32
D = 128
S = 16
F = 64
CUTOFF = 0.5
E = N * NBR
NH = NBR // 2
ZR = 1024
NT = N + ZR

NC, NS = 2, 16
NW = NC * NS
EPW = E // NW
CH = 400


def _ssp(v):
    return jnp.maximum(v, 0.0) + jnp.log1p(jnp.exp(-jnp.abs(v))) - _LN2


def _in2f_body(x_ref, w_ref, o_ref):
    o_ref[0:N, :] = jnp.dot(x_ref[...], w_ref[...],
                            preferred_element_type=jnp.float32)
    o_ref[N:NT, :] = jnp.zeros((NT - N, F), jnp.float32)


def _in2f(x2, w_in):
    return pl.pallas_call(
        _in2f_body,
        out_shape=jax.ShapeDtypeStruct((NT, F), jnp.float32),
    )(x2, w_in)


def _sc_gather(table, idx):
    mesh = plsc.VectorSubcoreMesh(core_axis_name="c", subcore_axis_name="s")

    @functools.partial(
        pl.kernel,
        out_type=jax.ShapeDtypeStruct((NH, N, D), jnp.float32),
        mesh=mesh,
        scratch_types=[
            pltpu.VMEM((EPW,), jnp.int32),
            pltpu.VMEM((CH, F), jnp.float32),
            pltpu.VMEM((CH, F), jnp.float32),
            pltpu.SemaphoreType.DMA,
            pltpu.SemaphoreType.DMA,
        ],
        compiler_params=pltpu.CompilerParams(use_tc_tiling_on_sc=False),
    )
    def k(table_hbm, idx_hbm, out_hbm, idx_v, rows_a, rows_b, sem_g, sem_w):
        wid = lax.axis_index("s") * NC + lax.axis_index("c")
        half = wid // NH
        slot = wid % NH
        pltpu.sync_copy(idx_hbm.at[pl.ds(wid * EPW, EPW)], idx_v)
        bufs = (rows_a, rows_b)
        nch = EPW // CH

        def gather(i):
            return pltpu.async_copy(
                table_hbm.at[idx_v.at[pl.ds(i * CH, CH)]],
                bufs[i % 2], sem_g)

        def write(i):
            return pltpu.async_copy(
                bufs[i % 2],
                out_hbm.at[slot, pl.ds(i * CH, CH), pl.ds(half * F, F)],
                sem_w)

        g = gather(0)
        w_prev = None
        for i in range(nch):
            g.wait()
            if w_prev is not None:
                w_prev.wait()
            if i + 1 < nch:
                g = gather(i + 1)
            w = write(i)
            w_prev = w if i + 1 < nch else None
            if i + 1 >= nch:
                w.wait()

    return k(table, idx)


def _filt_body(fa_ref, fb_ref, w1t_ref, b1_ref, w2t_ref, b2_ref, o_ref):
    ws = []
    for f_ref in (fa_ref, fb_ref):
        h = jnp.dot(w1t_ref[...], f_ref[0, 0],
                    preferred_element_type=jnp.float32) + b1_ref[...]
        ws.append(jnp.dot(w2t_ref[...], _ssp(h),
                          preferred_element_type=jnp.float32) + b2_ref[...])
    wcat = jnp.concatenate(ws, axis=0)
    o_ref[...] = wcat.T[None]


def _filt(ft, w1t, b1c, w2t, b2c):
    full2 = lambda j: (0, 0)
    return pl.pallas_call(
        _filt_body,
        grid=(NH,),
        in_specs=[
            pl.BlockSpec((1, 1, S, N), lambda j: (0, j, 0, 0)),
            pl.BlockSpec((1, 1, S, N), lambda j: (0, j + NH, 0, 0)),
            pl.BlockSpec((F, S), full2),
            pl.BlockSpec((F, 1), full2),
            pl.BlockSpec((F, F), full2),
            pl.BlockSpec((F, 1), full2),
        ],
        out_specs=pl.BlockSpec((1, N, D), lambda j: (j, 0, 0)),
        out_shape=jax.ShapeDtypeStruct((NH, N, D), jnp.float32),
    )(ft, ft, w1t, b1c, w2t, b2c)


def _tail_body(g_ref, w_ref, wo2_ref, bo_ref, wd_ref, bd_ref, o_ref):
    p = (g_ref[...] * w_ref[...]).sum(axis=0)
    y = _ssp(jnp.dot(p, wo2_ref[...],
                     preferred_element_type=jnp.float32) + bo_ref[...])
    o_ref[...] = jnp.dot(y, wd_ref[...],
                         preferred_element_type=jnp.float32) + bd_ref[...]


def _tail(gp, wp, wo2, bo, wd, bd, ta):
    grid = N // ta
    full = lambda i: (0, 0)
    return pl.pallas_call(
        functools.partial(_tail_body),
        grid=(grid,),
        in_specs=[
            pl.BlockSpec((NH, ta, D), lambda i: (0, i, 0)),
            pl.BlockSpec((NH, ta, D), lambda i: (0, i, 0)),
            pl.BlockSpec((D, D), full),
            pl.BlockSpec((1, D), full),
            pl.BlockSpec((D, D), full),
            pl.BlockSpec((1, D), full),
        ],
        out_specs=pl.BlockSpec((ta, D), lambda i: (i, 0)),
        out_shape=jax.ShapeDtypeStruct((N, D), jnp.float32),
    )(gp, wp, wo2, bo, wd, bd)


def kernel(x, r_ij, neighbors, neighbor_mask, f_ij,
           W1, b1, W2, b2, W_in, W_out, b_out, W_d, b_d):
    x2 = x.reshape(N, D)
    keep = (r_ij <= CUTOFF) & (neighbor_mask != 0)
    ii = lax.broadcasted_iota(jnp.int32, (1, N, NBR), 1)
    jj = lax.broadcasted_iota(jnp.int32, (1, N, NBR), 2)
    spread = N + ((ii * NBR + jj) & (ZR - 1))
    idx3 = jnp.where(keep, neighbors.astype(jnp.int32), spread)
    idx = jnp.swapaxes(idx3, 1, 2).reshape(E)
    ft = jnp.transpose(f_ij, (0, 2, 3, 1))

    w1t = W1.T
    w2t = W2.T
    b1c = b1.reshape(F, 1)
    b2c = b2.reshape(F, 1)
    wo2 = jnp.concatenate([W_out, W_out], axis=0)

    y = _in2f(x2, W_in)
    gp = _sc_gather(y, idx)
    wp = _filt(ft, w1t, b1c, w2t, b2c)
    v = _tail(gp, wp, wo2,
              b_out.reshape(1, D), W_d, b_d.reshape(1, D), ta=400)
    return v.reshape(1, N, D)
